# Initial kernel scaffold; baseline (speedup 1.0000x reference)
import jax
import jax.numpy as jnp
from jax.experimental import pallas as pl


def kernel(xytp, params):
    raise NotImplementedError("write your pallas kernel here")


# speedup vs baseline: 7.0040x; 7.0040x over previous
import functools
import math

import jax
import jax.numpy as jnp
from jax import lax
from jax.experimental import pallas as pl
from jax.experimental.pallas import tpu as pltpu
from jax.experimental.pallas import tpu_sc as plsc

HEIGHT = 260
B = 2
N = 4096
KB = 9
KL = 16
ROWS = 256
NT = N // ROWS
TD = 128


def _ln(x, g, b, eps=1e-5):
    m = jnp.mean(x, axis=-1, keepdims=True)
    v = jnp.mean((x - m) ** 2, axis=-1, keepdims=True)
    return (x - m) / jnp.sqrt(v + eps) * g + b


def _gelu(x):
    c = math.sqrt(2.0 / math.pi)
    return 0.5 * x * (1.0 + jnp.tanh(c * (x + 0.044715 * (x * x * x))))


def _sigmoid(x):
    return 1.0 / (1.0 + jnp.exp(-x))


def _softplus(x):
    return jnp.maximum(x, 0.0) + jnp.log1p(jnp.exp(-jnp.abs(x)))


def _full_spec(shape):
    nd = len(shape)
    return pl.BlockSpec(shape, lambda *_: (0,) * nd)


def _knn_body(x_ref, xt_ref, ball_ref, knn_ref):
    i = pl.program_id(1)
    r2 = (5.0 / HEIGHT) ** 2
    xr = x_ref[0]
    xt = xt_ref[0]
    r3 = xr[:, 0:3]
    c3 = xt[0:3, :]
    aa3 = jnp.sum(r3 * r3, axis=1, keepdims=True)
    bb3 = jnp.sum(c3 * c3, axis=0, keepdims=True)
    d2 = (aa3 + bb3) - 2.0 * jnp.dot(r3, c3)
    r2_ = xr[:, 1:3]
    c2_ = xt[1:3, :]
    aa2 = jnp.sum(r2_ * r2_, axis=1, keepdims=True)
    bb2 = jnp.sum(c2_ * c2_, axis=0, keepdims=True)
    d2xy = (aa2 + bb2) - 2.0 * jnp.dot(r2_, c2_)
    col = lax.broadcasted_iota(jnp.int32, (ROWS, N), 1)
    rowid = i * ROWS + lax.broadcasted_iota(jnp.int32, (ROWS, 1), 0)
    keys = jnp.where(d2xy <= r2, col, N)
    for k in range(KB):
        m = jnp.min(keys, axis=1, keepdims=True)
        ball_ref[0, :, k : k + 1] = jnp.where(m >= N, rowid, m)
        keys = jnp.where(keys == m, N, keys)
    for k in range(KB, 16):
        ball_ref[0, :, k : k + 1] = rowid
    dk = d2
    for k in range(KL):
        m = jnp.min(dk, axis=1, keepdims=True)
        cand = jnp.where(dk == m, col, N)
        c = jnp.min(cand, axis=1, keepdims=True)
        knn_ref[0, :, k : k + 1] = c
        dk = jnp.where(col == c, jnp.float32(jnp.inf), dk)


def _tc_knn(xytp, xytp_t):
    return pl.pallas_call(
        _knn_body,
        grid=(B, NT),
        in_specs=[
            pl.BlockSpec((1, ROWS, 4), lambda b, i: (b, i, 0)),
            pl.BlockSpec((1, 4, N), lambda b, i: (b, 0, 0)),
        ],
        out_specs=[
            pl.BlockSpec((1, ROWS, 16), lambda b, i: (b, i, 0)),
            pl.BlockSpec((1, ROWS, 16), lambda b, i: (b, i, 0)),
        ],
        out_shape=[
            jax.ShapeDtypeStruct((B, N, 16), jnp.int32),
            jax.ShapeDtypeStruct((B, N, 16), jnp.int32),
        ],
    )(xytp, xytp_t)


def _sc_gather(table, idx):
    M = idx.shape[0]
    D = table.shape[1]
    chunk = 128
    nw = 32
    per_w = M // nw
    nch = per_w // chunk
    mesh = plsc.VectorSubcoreMesh(core_axis_name="c", subcore_axis_name="s")

    @functools.partial(
        pl.kernel,
        out_type=jax.ShapeDtypeStruct((M, D), jnp.float32),
        mesh=mesh,
        scratch_types=[
            pltpu.VMEM((chunk,), jnp.int32),
            pltpu.VMEM((chunk, D), jnp.float32),
            pltpu.SemaphoreType.DMA,
        ],
    )
    def gk(tab_hbm, idx_hbm, out_hbm, idx_v, rows_v, sem):
        wid = lax.axis_index("s") * 2 + lax.axis_index("c")
        base = wid * per_w

        def body(c, carry):
            off = base + c * chunk
            pltpu.sync_copy(idx_hbm.at[pl.ds(off, chunk)], idx_v)
            pltpu.async_copy(tab_hbm.at[idx_v], rows_v, sem).wait()
            pltpu.sync_copy(rows_v, out_hbm.at[pl.ds(off, chunk)])
            return carry

        lax.fori_loop(0, nch, body, 0)

    return gk(table, idx)


def _embed_body(x_ref, g_ref, spw1, spb1, spw2, spb2, tepw, tepb, tenw, tenb,
                projw, elg, elb, n1g, n1b, ltw, ltb, pew1,
                main_ref, table_ref):
    xr = x_ref[0]
    g = g_ref[0]
    xy = xr[:, 1:3]
    xyrep = jnp.concatenate([xy] * KB, axis=1)
    delta = xyrep - g
    h = _gelu(jnp.dot(delta, spw1[...]) + spb1[...])
    fsp = jnp.dot(h, spw2[...]) + spb2[...]
    ts = xr[:, 0:1]
    pn = xr[:, 3:4]
    fte = (ts * tepw[...] + tepb[...]) * pn + (ts * tenw[...] + tenb[...]) * (1.0 - pn)
    pw = projw[...]
    f = jnp.dot(fsp, pw[0:32]) + jnp.dot(fte, pw[32:64])
    feats = _ln(f, elg[...], elb[...])
    x1 = _ln(feats, n1g[...], n1b[...])
    lt = jnp.dot(x1, ltw[...]) + ltb[...]
    u = jnp.dot(xr, pew1[...])
    main_ref[0, :, 0:32] = feats
    main_ref[0, :, 32:64] = lt[:, 0:32]
    main_ref[0, :, 64:68] = u
    table_ref[0, :, 0:4] = u
    table_ref[0, :, 4:36] = lt[:, 32:64]
    table_ref[0, :, 36:68] = lt[:, 64:96]


def _tc_embed(xytp, g1, pp):
    params = [pp['sp_w1'], pp['sp_b1'], pp['sp_w2'], pp['sp_b2'],
              pp['te_pos_w'], pp['te_pos_b'], pp['te_neg_w'], pp['te_neg_b'],
              pp['proj_w'], pp['emb_ln_g'], pp['emb_ln_b'],
              pp['norm1_g'], pp['norm1_b'], pp['lt_w'], pp['lt_b'], pp['pe_w1']]
    return pl.pallas_call(
        _embed_body,
        grid=(B, NT),
        in_specs=[
            pl.BlockSpec((1, ROWS, 4), lambda b, i: (b, i, 0)),
            pl.BlockSpec((1, ROWS, 2 * KB), lambda b, i: (b, i, 0)),
        ] + [_full_spec(p.shape) for p in params],
        out_specs=[
            pl.BlockSpec((1, ROWS, 128), lambda b, i: (b, i, 0)),
            pl.BlockSpec((1, ROWS, TD), lambda b, i: (b, i, 0)),
        ],
        out_shape=[
            jax.ShapeDtypeStruct((B, N, 128), jnp.float32),
            jax.ShapeDtypeStruct((B, N, TD), jnp.float32),
        ],
    )(xytp, g1, *params)


def _stats_body(gu_ref, urep_ref, b1rep_ref, out_ref):
    gu = gu_ref[...]
    t1 = (urep_ref[...] - gu) + b1rep_ref[...]

    def fold(x):
        s = x[:, 0:4]
        for k in range(1, KL):
            s = s + x[:, 4 * k : 4 * k + 4]
        return s

    cnt = float(B * N * KL)
    s1 = jnp.sum(t1, axis=0, keepdims=True)
    m = fold(s1) / cnt
    s2 = jnp.sum(t1 * t1, axis=0, keepdims=True)
    v = fold(s2) / cnt - m * m
    out_ref[0:1, 0:4] = m
    out_ref[1:2, 0:4] = v


def _tc_stats(gu, u, b1):
    return pl.pallas_call(
        _stats_body,
        in_specs=[_full_spec(gu.shape), _full_spec(u.shape), _full_spec(b1.shape)],
        out_specs=_full_spec((8, 128)),
        out_shape=jax.ShapeDtypeStruct((8, 128), jnp.float32),
    )(gu, u, b1)


def _attn_body(m_ref, g_ref, st_ref, b1, bng, bnb, pew2, peb2, llg, llb,
               out_ref):
    mn = m_ref[0]
    varphi = mn[:, 32:64]
    u = mn[:, 64:68]
    g = g_ref[0]
    mm = st_ref[0:1, 0:4]
    vv = st_ref[1:2, 0:4]
    inv = 1.0 / jnp.sqrt(vv + 1e-5)
    scale = 1.0 / math.sqrt(32.0)
    tks = []
    vals = []
    for k in range(KL):
        gk = g[:, k * TD : (k + 1) * TD]
        gu = gk[:, 0:4]
        psi = gk[:, 4:36]
        alpha = gk[:, 36:68]
        t1 = u - gu + b1[...]
        t1 = (t1 - mm) * inv * bng[...] + bnb[...]
        t1 = jnp.maximum(t1, 0.0)
        delta = jnp.dot(t1, pew2[...]) + peb2[...]
        tks.append(_ln(varphi - psi + delta, llg[...], llb[...]) * scale)
        vals.append(alpha + delta)
    mx = tks[0]
    for k in range(1, KL):
        mx = jnp.maximum(mx, tks[k])
    ssum = jnp.zeros((ROWS, 32), jnp.float32)
    osum = jnp.zeros((ROWS, 32), jnp.float32)
    for k in range(KL):
        e = jnp.exp(tks[k] - mx)
        ssum = ssum + e
        osum = osum + e * vals[k]
    out_ref[0] = osum / ssum


def _tc_attn(main, gtab, stats, pp):
    params = [pp['pe_b1'], pp['pe_bn_g'], pp['pe_bn_b'], pp['pe_w2'],
              pp['pe_b2'], pp['local_ln_g'], pp['local_ln_b']]
    return pl.pallas_call(
        _attn_body,
        grid=(B, NT),
        in_specs=[
            pl.BlockSpec((1, ROWS, 128), lambda b, i: (b, i, 0)),
            pl.BlockSpec((1, ROWS, KL * TD), lambda b, i: (b, i, 0)),
            _full_spec((8, 128)),
        ] + [_full_spec(p.shape) for p in params],
        out_specs=pl.BlockSpec((1, ROWS, 32), lambda b, i: (b, i, 0)),
        out_shape=jax.ShapeDtypeStruct((B, N, 32), jnp.float32),
    )(main, gtab, stats, *params)


def _mamba_body(a_ref, f_ref, ipw, cwT, cb, xwdt, xwB, xwC, dtw, dtb, alogT,
                dp, opw, n2g, n2b, w1, b1, w2, b2, hwT, hb,
                out_ref, dts, xcs, bs, cs, ys):
    AT = -jnp.exp(alogT[...])
    ones_row = jnp.ones((1, 64), jnp.float32)
    for bb in range(B):
        xb = a_ref[bb]
        xz = jnp.dot(xb, ipw[...])
        xc = xz[:, 0:64]
        zg = xz[:, 64:128]
        cw = cwT[...]
        acc = xc * cw[3:4]
        for j in range(1, 4):
            sh = jnp.concatenate(
                [jnp.zeros((j, 64), jnp.float32), xc[0 : N - j]], axis=0)
            acc = acc + sh * cw[3 - j : 4 - j]
        xconv = acc + cb[...]
        xconv = xconv * _sigmoid(xconv)
        dtr = jnp.dot(xconv, xwdt[...])
        dt = _softplus(jnp.dot(dtr, dtw[...]) + dtb[...])
        Bm = jnp.dot(xconv, xwB[...])
        Cm = jnp.dot(xconv, xwC[...])
        dts[...] = dt
        xcs[...] = xconv
        bs[...] = Bm
        cs[...] = Cm

        def step(t, h):
            dtr_ = dts[pl.ds(t, 1), :]
            xtr = xcs[pl.ds(t, 1), :]
            br = bs[pl.ds(t, 1), :]
            cr = cs[pl.ds(t, 1), :]
            bmat = lax.dot_general(br, ones_row, (((0,), (0,)), ((), ())),
                                   preferred_element_type=jnp.float32)
            h = jnp.exp(AT * dtr_) * h + bmat * (dtr_ * xtr)
            ys[pl.ds(t, 1), :] = jnp.dot(cr, h,
                                         preferred_element_type=jnp.float32)
            return h

        lax.fori_loop(0, N, step, jnp.zeros((64, 64), jnp.float32))
        y = ys[...]
        y = y + xconv * dp[...]
        y = y * (zg * _sigmoid(zg))
        attn = jnp.dot(y, opw[...])
        fa = attn + f_ref[bb]
        h2 = _ln(fa, n2g[...], n2b[...])
        mlp = jnp.dot(_gelu(jnp.dot(h2, w1[...]) + b1[...]), w2[...]) + b2[...]
        fa = mlp + fa
        o = jnp.sum(fa * hwT[...], axis=1, keepdims=True) + hb[...]
        out_ref[bb] = jnp.broadcast_to(o, (N, 8))


def _tc_mamba(attn_l, feats, pp):
    params = [pp['in_proj_w'], pp['conv_w_T'], pp['conv_b'],
              pp['xw_dt'], pp['xw_B'], pp['xw_C'],
              pp['dt_proj_w'], pp['dt_proj_b'], pp['A_log_T'], pp['Dp'],
              pp['out_proj_w'], pp['norm2_g'], pp['norm2_b'],
              pp['mlp_w1'], pp['mlp_b1'], pp['mlp_w2'], pp['mlp_b2'],
              pp['head_w_T'], pp['head_b']]
    return pl.pallas_call(
        _mamba_body,
        in_specs=[_full_spec((B, N, 32)), _full_spec((B, N, 32))]
        + [_full_spec(p.shape) for p in params],
        out_specs=_full_spec((B, N, 8)),
        out_shape=jax.ShapeDtypeStruct((B, N, 8), jnp.float32),
        scratch_shapes=[
            pltpu.VMEM((N, 64), jnp.float32),
            pltpu.VMEM((N, 64), jnp.float32),
            pltpu.VMEM((N, 64), jnp.float32),
            pltpu.VMEM((N, 64), jnp.float32),
            pltpu.VMEM((N, 64), jnp.float32),
        ],
    )(attn_l, feats, *params)


def _row(v):
    return v.reshape(1, -1)


def kernel(xytp, params):
    p = dict(params)
    for k in ['sp_b1', 'sp_b2', 'te_pos_b', 'te_neg_b', 'emb_ln_g', 'emb_ln_b',
              'norm1_g', 'norm1_b', 'lt_b', 'pe_b1', 'pe_bn_g', 'pe_bn_b',
              'pe_b2', 'local_ln_g', 'local_ln_b', 'conv_b', 'dt_proj_b',
              'Dp', 'norm2_g', 'norm2_b', 'mlp_b1', 'mlp_b2']:
        p[k] = _row(p[k])
    p['conv_w_T'] = p['conv_w'].T
    p['xw_dt'] = p['x_proj_w'][:, 0:2]
    p['xw_B'] = p['x_proj_w'][:, 2:66]
    p['xw_C'] = p['x_proj_w'][:, 66:130]
    p['head_w_T'] = p['head_w'].T
    p['A_log_T'] = p['A_log'].T
    p['head_b'] = _row(p['head_b'])

    xytp_t = xytp.transpose(0, 2, 1)
    ball, knn = _tc_knn(xytp, xytp_t)

    offs = (jnp.arange(B, dtype=jnp.int32) * N)[:, None, None]
    g1_idx = (ball[..., :KB] + offs).reshape(-1)
    xy_pad = jnp.pad(xytp[..., 1:3].reshape(B * N, 2), ((0, 0), (0, 126)))
    gxy = _sc_gather(xy_pad, g1_idx)
    g1 = gxy.reshape(B, N, KB, 128)[..., :2].reshape(B, N, KB * 2)

    main, table = _tc_embed(xytp, g1, p)

    g2_idx = (knn + offs).reshape(-1)
    gtab = _sc_gather(table.reshape(B * N, TD), g2_idx)

    stats = _tc_stats(gtab[:, 0:4].reshape(B * N, 4 * KL),
                      jnp.tile(main[..., 64:68].reshape(B * N, 4), (1, KL)),
                      jnp.tile(p['pe_b1'], (1, KL)))
    attn_l = _tc_attn(main, gtab.reshape(B, N, KL * TD), stats, p)
    outp = _tc_mamba(attn_l, main[..., 0:32], p)
    return outp[..., 0:1]
